# Initial kernel scaffold; baseline (speedup 1.0000x reference)
import jax
import jax.numpy as jnp
from jax.experimental import pallas as pl


def kernel(x, q_param, W_kv, W_proj, b_proj, ln_w, ln_b, mlp_w1, mlp_b1, mlp_w2, mlp_b2, perm1, idxs):
    raise NotImplementedError("write your pallas kernel here")


# speedup vs baseline: 8.5523x; 8.5523x over previous
import functools

import jax
import jax.numpy as jnp
from jax import lax
from jax.experimental import pallas as pl
from jax.experimental.pallas import tpu as pltpu
from jax.experimental.pallas import tpu_sc as plsc

_B, _N, _C = 8, 3136, 384
_H, _Dh = 8, 48
_M = 196
_EPS = 1e-6
_NEG = -1e30
_SCALE = 1.0 / (48.0 ** 0.5)
_RAD = 342
_PER = 56


def _sc_gather_rows(table, flat_idx):
    P = flat_idx.shape[0]
    D = table.shape[1]
    info = plsc.get_sparse_core_info()
    nw = info.num_cores * info.num_subcores
    per_w = P // nw
    mesh = plsc.VectorSubcoreMesh(core_axis_name="c", subcore_axis_name="s")

    @functools.partial(
        pl.kernel,
        mesh=mesh,
        out_type=jax.ShapeDtypeStruct((P, D), jnp.float32),
        scratch_types=[
            pltpu.VMEM((per_w,), jnp.int32),
            pltpu.VMEM((per_w, D), jnp.float32),
            pltpu.SemaphoreType.DMA,
        ],
    )
    def gather_kernel(table_hbm, idx_hbm, out_hbm, idx_v, rows_v, sem):
        wid = lax.axis_index("s") * info.num_cores + lax.axis_index("c")
        base = wid * per_w
        pltpu.sync_copy(idx_hbm.at[pl.ds(base, per_w)], idx_v)
        pltpu.async_copy(table_hbm.at[idx_v], rows_v, sem).wait()
        pltpu.sync_copy(rows_v, out_hbm.at[pl.ds(base, per_w)])

    return gather_kernel(table, flat_idx)


def _head_body(xg_ref, q_ref, wkv_ref, w1_ref, b1_ref, w2_ref, b2_ref,
               lnw_ref, lnb_ref, idxs_ref, qc_ref, cidx_ref):
    lnw = lnw_ref[...]
    lnb = lnb_ref[...]
    wkv = wkv_ref[...]

    def norm(xr):
        mu = jnp.mean(xr, axis=1, keepdims=True)
        xc = xr - mu
        var = jnp.mean(xc * xc, axis=1, keepdims=True)
        return xc * lax.rsqrt(var + _EPS) * lnw + lnb

    na = norm(xg_ref[0, 0])
    nb = norm(xg_ref[0, 1])
    kvg = lax.dot_general(na, wkv, (((1,), (1,)), ((), ())),
                          preferred_element_type=jnp.float32)
    k_idx = lax.dot_general(nb, wkv[:_C, :], (((1,), (1,)), ((), ())),
                            preferred_element_type=jnp.float32)

    q = q_ref[...]
    outs = []
    for h in range(_H):
        sl = slice(h * _Dh, (h + 1) * _Dh)
        k_h = kvg[:, h * _Dh:(h + 1) * _Dh]
        v_h = kvg[:, _C + h * _Dh: _C + (h + 1) * _Dh]
        l = lax.dot_general(q[:, sl], k_h, (((1,), (1,)), ((), ())),
                            preferred_element_type=jnp.float32) * _SCALE
        m_ = jnp.max(l, axis=1, keepdims=True)
        p = jnp.exp(l - m_)
        s_ = jnp.sum(p, axis=1, keepdims=True)
        outs.append(lax.dot_general(p, v_h, (((1,), (0,)), ((), ())),
                                    preferred_element_type=jnp.float32) / s_)
    q_att = jnp.concatenate(outs, axis=1)

    hdn = lax.dot_general(q_att, w1_ref[...], (((1,), (1,)), ((), ())),
                          preferred_element_type=jnp.float32) + b1_ref[...]
    hdn = 0.5 * hdn * (1.0 + lax.erf(hdn * (2.0 ** -0.5)))
    qc = lax.dot_general(hdn, w2_ref[...], (((1,), (1,)), ((), ())),
                         preferred_element_type=jnp.float32) + b2_ref[...]
    qc_ref[0] = qc

    sc = lax.dot_general(qc, k_idx, (((1,), (1,)), ((), ())),
                         preferred_element_type=jnp.float32)
    rmax = jnp.max(sc, axis=1, keepdims=True)
    jj = lax.broadcasted_iota(jnp.int32, (_M, _M), 1)
    cand = jnp.where(sc == rmax, jj, _M)
    cent = jnp.min(cand, axis=1, keepdims=True)
    cidx = jnp.sum(jnp.where(jj == cent, idxs_ref[...], 0),
                   axis=1, keepdims=True)
    cidx_ref[0] = cidx.astype(jnp.int32)


def _main_body(x_ref, qc_ref, cidx_ref, wkv_ref, lnw_ref, lnb_ref,
               wproj_ref, bproj_ref, out_ref, kv_s):
    lnw = lnw_ref[...]
    lnb = lnb_ref[...]
    xb = x_ref[0]
    mu = jnp.mean(xb, axis=1, keepdims=True)
    xc = xb - mu
    var = jnp.mean(xc * xc, axis=1, keepdims=True)
    nx = xc * lax.rsqrt(var + _EPS) * lnw + lnb
    kv_s[...] = lax.dot_general(nx, wkv_ref[...], (((1,), (1,)), ((), ())),
                                preferred_element_type=jnp.float32)

    c = cidx_ref[0]
    n_io = lax.broadcasted_iota(jnp.int32, (_M, _N), 1)
    d = n_io - c
    valid = (jnp.abs(d) <= _RAD) & (lax.rem(d + 3198, _PER) <= 12)
    valid = valid | ((n_io == 0) & (c <= _RAD))
    valid = valid | ((n_io == _N - 1) & (c >= _N - 1 - _RAD))

    qc = qc_ref[0]
    outs = []
    for h in range(_H):
        k_h = kv_s[:, h * _Dh:(h + 1) * _Dh]
        v_h = kv_s[:, _C + h * _Dh: _C + (h + 1) * _Dh]
        l = lax.dot_general(qc[:, h * _Dh:(h + 1) * _Dh], k_h,
                            (((1,), (1,)), ((), ())),
                            preferred_element_type=jnp.float32) * _SCALE
        l = jnp.where(valid, l, _NEG)
        m_ = jnp.max(l, axis=1, keepdims=True)
        p = jnp.exp(l - m_)
        s_ = jnp.sum(p, axis=1, keepdims=True)
        outs.append(lax.dot_general(p, v_h, (((1,), (0,)), ((), ())),
                                    preferred_element_type=jnp.float32) / s_)
    att = jnp.concatenate(outs, axis=1)
    out_ref[0] = lax.dot_general(att, wproj_ref[...], (((1,), (1,)), ((), ())),
                                 preferred_element_type=jnp.float32) + bproj_ref[...]


def kernel(x, q_param, W_kv, W_proj, b_proj, ln_w, ln_b,
           mlp_w1, mlp_b1, mlp_w2, mlp_b2, perm1, idxs):
    Bs, Ns, Cs = x.shape

    idx2 = jnp.concatenate([perm1, idxs]).astype(jnp.int32)
    flat = (jnp.arange(Bs, dtype=jnp.int32)[:, None] * Ns
            + idx2[None, :]).reshape(-1)
    pad = (-flat.shape[0]) % 256
    flat = jnp.concatenate([flat, jnp.zeros((pad,), jnp.int32)])
    rows = _sc_gather_rows(x.reshape(Bs * Ns, Cs), flat)
    xg = rows[: Bs * 2 * _M].reshape(Bs, 2, _M, Cs)

    q2 = q_param.reshape(_M, Cs)
    b1 = mlp_b1.reshape(1, 2 * Cs)
    b2 = mlp_b2.reshape(1, Cs)
    lnw = ln_w.reshape(1, Cs)
    lnb = ln_b.reshape(1, Cs)
    bp = b_proj.reshape(1, Cs)
    idxs_row = idxs.reshape(1, _M).astype(jnp.int32)

    qc, cidx = pl.pallas_call(
        _head_body,
        grid=(Bs,),
        in_specs=[
            pl.BlockSpec((1, 2, _M, Cs), lambda b: (b, 0, 0, 0)),
            pl.BlockSpec((_M, Cs), lambda b: (0, 0)),
            pl.BlockSpec((2 * Cs, Cs), lambda b: (0, 0)),
            pl.BlockSpec((2 * Cs, Cs), lambda b: (0, 0)),
            pl.BlockSpec((1, 2 * Cs), lambda b: (0, 0)),
            pl.BlockSpec((Cs, 2 * Cs), lambda b: (0, 0)),
            pl.BlockSpec((1, Cs), lambda b: (0, 0)),
            pl.BlockSpec((1, Cs), lambda b: (0, 0)),
            pl.BlockSpec((1, Cs), lambda b: (0, 0)),
            pl.BlockSpec((1, _M), lambda b: (0, 0)),
        ],
        out_specs=[
            pl.BlockSpec((1, _M, Cs), lambda b: (b, 0, 0)),
            pl.BlockSpec((1, _M, 1), lambda b: (b, 0, 0)),
        ],
        out_shape=[
            jax.ShapeDtypeStruct((Bs, _M, Cs), jnp.float32),
            jax.ShapeDtypeStruct((Bs, _M, 1), jnp.int32),
        ],
    )(xg, q2, W_kv, mlp_w1, b1, mlp_w2, b2, lnw, lnb, idxs_row)

    out = pl.pallas_call(
        _main_body,
        grid=(Bs,),
        in_specs=[
            pl.BlockSpec((1, Ns, Cs), lambda b: (b, 0, 0)),
            pl.BlockSpec((1, _M, Cs), lambda b: (b, 0, 0)),
            pl.BlockSpec((1, _M, 1), lambda b: (b, 0, 0)),
            pl.BlockSpec((2 * Cs, Cs), lambda b: (0, 0)),
            pl.BlockSpec((1, Cs), lambda b: (0, 0)),
            pl.BlockSpec((1, Cs), lambda b: (0, 0)),
            pl.BlockSpec((Cs, Cs), lambda b: (0, 0)),
            pl.BlockSpec((1, Cs), lambda b: (0, 0)),
        ],
        out_specs=pl.BlockSpec((1, _M, Cs), lambda b: (b, 0, 0)),
        out_shape=jax.ShapeDtypeStruct((Bs, _M, Cs), jnp.float32),
        scratch_shapes=[pltpu.VMEM((Ns, 2 * Cs), jnp.float32)],
    )(x, qc, cidx, W_kv, lnw, lnb, W_proj, bp)
    return out


# speedup vs baseline: 8.5568x; 1.0005x over previous
import functools

import jax
import jax.numpy as jnp
from jax import lax
from jax.experimental import pallas as pl
from jax.experimental.pallas import tpu as pltpu
from jax.experimental.pallas import tpu_sc as plsc

_B, _N, _C = 8, 3136, 384
_H, _Dh = 8, 48
_M = 196
_EPS = 1e-6
_NEG = -1e30
_SCALE = 1.0 / (48.0 ** 0.5)
_RAD = 342
_PER = 56


def _sc_gather_rows(table, flat_idx):
    P = flat_idx.shape[0]
    D = table.shape[1]
    info = plsc.get_sparse_core_info()
    nw = info.num_cores * info.num_subcores
    per_w = P // nw
    mesh = plsc.VectorSubcoreMesh(core_axis_name="c", subcore_axis_name="s")

    @functools.partial(
        pl.kernel,
        mesh=mesh,
        out_type=jax.ShapeDtypeStruct((P, D), jnp.float32),
        scratch_types=[
            pltpu.VMEM((per_w,), jnp.int32),
            pltpu.VMEM((per_w, D), jnp.float32),
            pltpu.SemaphoreType.DMA,
        ],
    )
    def gather_kernel(table_hbm, idx_hbm, out_hbm, idx_v, rows_v, sem):
        wid = lax.axis_index("s") * info.num_cores + lax.axis_index("c")
        base = wid * per_w
        pltpu.sync_copy(idx_hbm.at[pl.ds(base, per_w)], idx_v)
        pltpu.async_copy(table_hbm.at[idx_v], rows_v, sem).wait()
        pltpu.sync_copy(rows_v, out_hbm.at[pl.ds(base, per_w)])

    return gather_kernel(table, flat_idx)


def _head_body(xg_ref, q_ref, wkv_ref, w1_ref, b1_ref, w2_ref, b2_ref,
               lnw_ref, lnb_ref, idxs_ref, qc_ref, cidx_ref):
    lnw = lnw_ref[...]
    lnb = lnb_ref[...]
    wkv = wkv_ref[...]

    def norm(xr):
        mu = jnp.mean(xr, axis=1, keepdims=True)
        xc = xr - mu
        var = jnp.mean(xc * xc, axis=1, keepdims=True)
        return xc * lax.rsqrt(var + _EPS) * lnw + lnb

    na = norm(xg_ref[0, 0])
    nb = norm(xg_ref[0, 1])
    kvg = lax.dot_general(na, wkv, (((1,), (1,)), ((), ())),
                          preferred_element_type=jnp.float32)
    k_idx = lax.dot_general(nb, wkv[:_C, :], (((1,), (1,)), ((), ())),
                            preferred_element_type=jnp.float32)

    q = q_ref[...]
    outs = []
    for h in range(_H):
        sl = slice(h * _Dh, (h + 1) * _Dh)
        k_h = kvg[:, h * _Dh:(h + 1) * _Dh]
        v_h = kvg[:, _C + h * _Dh: _C + (h + 1) * _Dh]
        l = lax.dot_general(q[:, sl], k_h, (((1,), (1,)), ((), ())),
                            preferred_element_type=jnp.float32) * _SCALE
        m_ = jnp.max(l, axis=1, keepdims=True)
        p = jnp.exp(l - m_)
        s_ = jnp.sum(p, axis=1, keepdims=True)
        outs.append(lax.dot_general(p, v_h, (((1,), (0,)), ((), ())),
                                    preferred_element_type=jnp.float32) / s_)
    q_att = jnp.concatenate(outs, axis=1)

    hdn = lax.dot_general(q_att, w1_ref[...], (((1,), (1,)), ((), ())),
                          preferred_element_type=jnp.float32) + b1_ref[...]
    hdn = 0.5 * hdn * (1.0 + lax.erf(hdn * (2.0 ** -0.5)))
    qc = lax.dot_general(hdn, w2_ref[...], (((1,), (1,)), ((), ())),
                         preferred_element_type=jnp.float32) + b2_ref[...]
    qc_ref[0] = qc

    sc = lax.dot_general(qc, k_idx, (((1,), (1,)), ((), ())),
                         preferred_element_type=jnp.float32)
    rmax = jnp.max(sc, axis=1, keepdims=True)
    jj = lax.broadcasted_iota(jnp.int32, (_M, _M), 1)
    cand = jnp.where(sc == rmax, jj, _M)
    cent = jnp.min(cand, axis=1, keepdims=True)
    cidx = jnp.sum(jnp.where(jj == cent, idxs_ref[...], 0),
                   axis=1, keepdims=True)
    cidx_ref[0] = cidx.astype(jnp.int32)


def _main_body(x_ref, qc_ref, cidx_ref, wkv_ref, lnw_ref, lnb_ref,
               wproj_ref, bproj_ref, out_ref, k_s, v_s, bias_s):
    lnw = lnw_ref[...]
    lnb = lnb_ref[...]
    wkv = wkv_ref[...]
    xb = x_ref[0]
    mu = jnp.mean(xb, axis=1, keepdims=True)
    xc = xb - mu
    var = jnp.mean(xc * xc, axis=1, keepdims=True)
    nx = xc * lax.rsqrt(var + _EPS) * lnw + lnb
    k_s[...] = lax.dot_general(nx, wkv[:_C, :], (((1,), (1,)), ((), ())),
                               preferred_element_type=jnp.float32)
    v_s[...] = lax.dot_general(nx, wkv[_C:, :], (((1,), (1,)), ((), ())),
                               preferred_element_type=jnp.float32
                               ).astype(jnp.bfloat16)

    c = cidx_ref[0]
    n_io = lax.broadcasted_iota(jnp.int32, (_M, _N), 1)
    d = n_io - c
    valid = (jnp.abs(d) <= _RAD) & (lax.rem(d + 3198, _PER) <= 12)
    valid = valid | ((n_io == 0) & (c <= _RAD))
    valid = valid | ((n_io == _N - 1) & (c >= _N - 1 - _RAD))
    bias_s[...] = jnp.where(valid, 0.0, _NEG).astype(jnp.bfloat16)

    qcs = qc_ref[0] * _SCALE
    outs = []
    for h in range(_H):
        k_h = k_s[:, h * _Dh:(h + 1) * _Dh]
        v_h = v_s[:, h * _Dh:(h + 1) * _Dh]
        l = lax.dot_general(qcs[:, h * _Dh:(h + 1) * _Dh], k_h,
                            (((1,), (1,)), ((), ())),
                            preferred_element_type=jnp.float32)
        l = l + bias_s[...].astype(jnp.float32)
        m_ = jnp.max(l, axis=1, keepdims=True)
        p = jnp.exp(l - m_)
        s_ = jnp.sum(p, axis=1, keepdims=True)
        pb = p.astype(jnp.bfloat16)
        outs.append(lax.dot_general(pb, v_h, (((1,), (0,)), ((), ())),
                                    preferred_element_type=jnp.float32) / s_)
    att = jnp.concatenate(outs, axis=1)
    out_ref[0] = lax.dot_general(att, wproj_ref[...], (((1,), (1,)), ((), ())),
                                 preferred_element_type=jnp.float32) + bproj_ref[...]


def kernel(x, q_param, W_kv, W_proj, b_proj, ln_w, ln_b,
           mlp_w1, mlp_b1, mlp_w2, mlp_b2, perm1, idxs):
    Bs, Ns, Cs = x.shape

    idx2 = jnp.concatenate([perm1, idxs]).astype(jnp.int32)
    flat = (jnp.arange(Bs, dtype=jnp.int32)[:, None] * Ns
            + idx2[None, :]).reshape(-1)
    pad = (-flat.shape[0]) % 256
    flat = jnp.concatenate([flat, jnp.zeros((pad,), jnp.int32)])
    rows = _sc_gather_rows(x.reshape(Bs * Ns, Cs), flat)
    xg = rows[: Bs * 2 * _M].reshape(Bs, 2, _M, Cs)

    q2 = q_param.reshape(_M, Cs)
    b1 = mlp_b1.reshape(1, 2 * Cs)
    b2 = mlp_b2.reshape(1, Cs)
    lnw = ln_w.reshape(1, Cs)
    lnb = ln_b.reshape(1, Cs)
    bp = b_proj.reshape(1, Cs)
    idxs_row = idxs.reshape(1, _M).astype(jnp.int32)

    qc, cidx = pl.pallas_call(
        _head_body,
        grid=(Bs,),
        in_specs=[
            pl.BlockSpec((1, 2, _M, Cs), lambda b: (b, 0, 0, 0)),
            pl.BlockSpec((_M, Cs), lambda b: (0, 0)),
            pl.BlockSpec((2 * Cs, Cs), lambda b: (0, 0)),
            pl.BlockSpec((2 * Cs, Cs), lambda b: (0, 0)),
            pl.BlockSpec((1, 2 * Cs), lambda b: (0, 0)),
            pl.BlockSpec((Cs, 2 * Cs), lambda b: (0, 0)),
            pl.BlockSpec((1, Cs), lambda b: (0, 0)),
            pl.BlockSpec((1, Cs), lambda b: (0, 0)),
            pl.BlockSpec((1, Cs), lambda b: (0, 0)),
            pl.BlockSpec((1, _M), lambda b: (0, 0)),
        ],
        out_specs=[
            pl.BlockSpec((1, _M, Cs), lambda b: (b, 0, 0)),
            pl.BlockSpec((1, _M, 1), lambda b: (b, 0, 0)),
        ],
        out_shape=[
            jax.ShapeDtypeStruct((Bs, _M, Cs), jnp.float32),
            jax.ShapeDtypeStruct((Bs, _M, 1), jnp.int32),
        ],
    )(xg, q2, W_kv, mlp_w1, b1, mlp_w2, b2, lnw, lnb, idxs_row)

    out = pl.pallas_call(
        _main_body,
        grid=(Bs,),
        in_specs=[
            pl.BlockSpec((1, Ns, Cs), lambda b: (b, 0, 0)),
            pl.BlockSpec((1, _M, Cs), lambda b: (b, 0, 0)),
            pl.BlockSpec((1, _M, 1), lambda b: (b, 0, 0)),
            pl.BlockSpec((2 * Cs, Cs), lambda b: (0, 0)),
            pl.BlockSpec((1, Cs), lambda b: (0, 0)),
            pl.BlockSpec((1, Cs), lambda b: (0, 0)),
            pl.BlockSpec((Cs, Cs), lambda b: (0, 0)),
            pl.BlockSpec((1, Cs), lambda b: (0, 0)),
        ],
        out_specs=pl.BlockSpec((1, _M, Cs), lambda b: (b, 0, 0)),
        out_shape=jax.ShapeDtypeStruct((Bs, _M, Cs), jnp.float32),
        scratch_shapes=[pltpu.VMEM((Ns, Cs), jnp.float32),
                        pltpu.VMEM((Ns, Cs), jnp.bfloat16),
                        pltpu.VMEM((_M, Ns), jnp.bfloat16)],
    )(x, qc, cidx, W_kv, lnw, lnb, W_proj, bp)
    return out


# speedup vs baseline: 9.8389x; 1.1498x over previous
import functools

import jax
import jax.numpy as jnp
from jax import lax
from jax.experimental import pallas as pl
from jax.experimental.pallas import tpu as pltpu
from jax.experimental.pallas import tpu_sc as plsc

_B, _N, _C = 8, 3136, 384
_H, _Dh = 8, 48
_M = 196
_CHUNK = 416
_IDX_OFF = 200
_EPS = 1e-6
_NEG = -1e30
_SCALE = 1.0 / (48.0 ** 0.5)
_RAD = 342
_PER = 56


def _sc_gather_rows(table, flat_idx):
    P = flat_idx.shape[0]
    D = table.shape[1]
    info = plsc.get_sparse_core_info()
    nw = info.num_cores * info.num_subcores
    per_w = P // nw
    mesh = plsc.VectorSubcoreMesh(core_axis_name="c", subcore_axis_name="s")

    @functools.partial(
        pl.kernel,
        mesh=mesh,
        out_type=jax.ShapeDtypeStruct((P, D), jnp.float32),
        scratch_types=[
            pltpu.VMEM((per_w,), jnp.int32),
            pltpu.VMEM((per_w, D), jnp.float32),
            pltpu.SemaphoreType.DMA,
        ],
    )
    def gather_kernel(table_hbm, idx_hbm, out_hbm, idx_v, rows_v, sem):
        wid = lax.axis_index("s") * info.num_cores + lax.axis_index("c")
        base = wid * per_w
        pltpu.sync_copy(idx_hbm.at[pl.ds(base, per_w)], idx_v)
        pltpu.async_copy(table_hbm.at[idx_v], rows_v, sem).wait()
        pltpu.sync_copy(rows_v, out_hbm.at[pl.ds(base, per_w)])

    return gather_kernel(table, flat_idx)


def _fused_body(rows_ref, x_ref, q_ref, wkv_ref, w1_ref, b1_ref, w2_ref,
                b2_ref, lnw_ref, lnb_ref, idxs_ref, wproj_ref, bproj_ref,
                out_ref, k_s, v_s):
    lnw = lnw_ref[...]
    lnb = lnb_ref[...]
    wkv = wkv_ref[...]

    def norm(xr):
        mu = jnp.mean(xr, axis=1, keepdims=True)
        xc = xr - mu
        var = jnp.mean(xc * xc, axis=1, keepdims=True)
        return xc * lax.rsqrt(var + _EPS) * lnw + lnb

    na = norm(rows_ref[0:_M])
    nb = norm(rows_ref[_IDX_OFF:_IDX_OFF + _M])
    kvg = lax.dot_general(na, wkv, (((1,), (1,)), ((), ())),
                          preferred_element_type=jnp.float32)
    k_idx = lax.dot_general(nb, wkv[:_C, :], (((1,), (1,)), ((), ())),
                            preferred_element_type=jnp.float32)

    q = q_ref[...]
    outs = []
    for h in range(_H):
        sl = slice(h * _Dh, (h + 1) * _Dh)
        k_h = kvg[:, h * _Dh:(h + 1) * _Dh]
        v_h = kvg[:, _C + h * _Dh: _C + (h + 1) * _Dh]
        l = lax.dot_general(q[:, sl], k_h, (((1,), (1,)), ((), ())),
                            preferred_element_type=jnp.float32) * _SCALE
        m_ = jnp.max(l, axis=1, keepdims=True)
        p = jnp.exp(l - m_)
        s_ = jnp.sum(p, axis=1, keepdims=True)
        outs.append(lax.dot_general(p, v_h, (((1,), (0,)), ((), ())),
                                    preferred_element_type=jnp.float32) / s_)
    q_att = jnp.concatenate(outs, axis=1)

    hdn = lax.dot_general(q_att, w1_ref[...], (((1,), (1,)), ((), ())),
                          preferred_element_type=jnp.float32) + b1_ref[...]
    hdn = 0.5 * hdn * (1.0 + lax.erf(hdn * (2.0 ** -0.5)))
    qc = lax.dot_general(hdn, w2_ref[...], (((1,), (1,)), ((), ())),
                         preferred_element_type=jnp.float32) + b2_ref[...]

    sc = lax.dot_general(qc, k_idx, (((1,), (1,)), ((), ())),
                         preferred_element_type=jnp.float32)
    rmax = jnp.max(sc, axis=1, keepdims=True)
    jj = lax.broadcasted_iota(jnp.int32, (_M, _M), 1)
    cand = jnp.where(sc == rmax, jj, _M)
    cent = jnp.min(cand, axis=1, keepdims=True)
    c = jnp.sum(jnp.where(jj == cent, idxs_ref[...], 0),
                axis=1, keepdims=True)

    xb = x_ref[0]
    mu = jnp.mean(xb, axis=1, keepdims=True)
    xc = xb - mu
    var = jnp.mean(xc * xc, axis=1, keepdims=True)
    nx = xc * lax.rsqrt(var + _EPS) * lnw + lnb
    k_s[...] = lax.dot_general(nx, wkv[:_C, :], (((1,), (1,)), ((), ())),
                               preferred_element_type=jnp.float32)
    v_s[...] = lax.dot_general(nx, wkv[_C:, :], (((1,), (1,)), ((), ())),
                               preferred_element_type=jnp.float32
                               ).astype(jnp.bfloat16)

    n_row = lax.broadcasted_iota(jnp.int32, (1, _N), 1)
    a_row = lax.rem(n_row + 6, _PER)
    e = a_row - lax.rem(c, _PER)
    f = jnp.where(e < 0, e + _PER, e)
    d = n_row - c
    valid = (f <= 12) & (jnp.abs(d) <= _RAD)
    valid = valid | ((n_row == 0) & (c <= _RAD))
    valid = valid | ((n_row == _N - 1) & (c >= _N - 1 - _RAD))
    bias = jnp.where(valid, 0.0, _NEG)

    qcs = qc * _SCALE
    outs = []
    for h in range(_H):
        k_h = k_s[:, h * _Dh:(h + 1) * _Dh]
        v_h = v_s[:, h * _Dh:(h + 1) * _Dh]
        l = lax.dot_general(qcs[:, h * _Dh:(h + 1) * _Dh], k_h,
                            (((1,), (1,)), ((), ())),
                            preferred_element_type=jnp.float32) + bias
        m_ = jnp.max(l, axis=1, keepdims=True)
        p = jnp.exp(l - m_)
        s_ = jnp.sum(p, axis=1, keepdims=True)
        pb = p.astype(jnp.bfloat16)
        outs.append(lax.dot_general(pb, v_h, (((1,), (0,)), ((), ())),
                                    preferred_element_type=jnp.float32) / s_)
    att = jnp.concatenate(outs, axis=1)
    out_ref[0] = lax.dot_general(att, wproj_ref[...], (((1,), (1,)), ((), ())),
                                 preferred_element_type=jnp.float32) + bproj_ref[...]


def kernel(x, q_param, W_kv, W_proj, b_proj, ln_w, ln_b,
           mlp_w1, mlp_b1, mlp_w2, mlp_b2, perm1, idxs):
    Bs, Ns, Cs = x.shape

    chunk_idx = jnp.zeros((_CHUNK,), jnp.int32)
    chunk_idx = chunk_idx.at[0:_M].set(perm1.astype(jnp.int32))
    chunk_idx = chunk_idx.at[_IDX_OFF:_IDX_OFF + _M].set(idxs.astype(jnp.int32))
    flat = (jnp.arange(Bs, dtype=jnp.int32)[:, None] * Ns
            + chunk_idx[None, :]).reshape(-1)
    rows = _sc_gather_rows(x.reshape(Bs * Ns, Cs), flat)

    q2 = q_param.reshape(_M, Cs)
    b1 = mlp_b1.reshape(1, 2 * Cs)
    b2 = mlp_b2.reshape(1, Cs)
    lnw = ln_w.reshape(1, Cs)
    lnb = ln_b.reshape(1, Cs)
    bp = b_proj.reshape(1, Cs)
    idxs_row = idxs.reshape(1, _M).astype(jnp.int32)

    out = pl.pallas_call(
        _fused_body,
        grid=(Bs,),
        in_specs=[
            pl.BlockSpec((_CHUNK, Cs), lambda b: (b, 0)),
            pl.BlockSpec((1, Ns, Cs), lambda b: (b, 0, 0)),
            pl.BlockSpec((_M, Cs), lambda b: (0, 0)),
            pl.BlockSpec((2 * Cs, Cs), lambda b: (0, 0)),
            pl.BlockSpec((2 * Cs, Cs), lambda b: (0, 0)),
            pl.BlockSpec((1, 2 * Cs), lambda b: (0, 0)),
            pl.BlockSpec((Cs, 2 * Cs), lambda b: (0, 0)),
            pl.BlockSpec((1, Cs), lambda b: (0, 0)),
            pl.BlockSpec((1, Cs), lambda b: (0, 0)),
            pl.BlockSpec((1, Cs), lambda b: (0, 0)),
            pl.BlockSpec((1, _M), lambda b: (0, 0)),
            pl.BlockSpec((Cs, Cs), lambda b: (0, 0)),
            pl.BlockSpec((1, Cs), lambda b: (0, 0)),
        ],
        out_specs=pl.BlockSpec((1, _M, Cs), lambda b: (b, 0, 0)),
        out_shape=jax.ShapeDtypeStruct((Bs, _M, Cs), jnp.float32),
        scratch_shapes=[pltpu.VMEM((Ns, Cs), jnp.float32),
                        pltpu.VMEM((Ns, Cs), jnp.bfloat16)],
    )(rows, x, q2, W_kv, mlp_w1, b1, mlp_w2, b2, lnw, lnb, idxs_row,
      W_proj, bp)
    return out


# speedup vs baseline: 10.0845x; 1.0250x over previous
import functools

import jax
import jax.numpy as jnp
from jax import lax
from jax.experimental import pallas as pl
from jax.experimental.pallas import tpu as pltpu
from jax.experimental.pallas import tpu_sc as plsc

_B, _N, _C = 8, 3136, 384
_H, _Dh = 8, 48
_M = 196
_CHUNK = 416
_IDX_OFF = 200
_EPS = 1e-6
_NEG = -1e30
_SCALE = 1.0 / (48.0 ** 0.5)
_RAD = 342
_PER = 56


def _sc_gather_rows(table, flat_idx):
    P = flat_idx.shape[0]
    D = table.shape[1]
    info = plsc.get_sparse_core_info()
    nw = info.num_cores * info.num_subcores
    per_w = P // nw
    mesh = plsc.VectorSubcoreMesh(core_axis_name="c", subcore_axis_name="s")

    @functools.partial(
        pl.kernel,
        mesh=mesh,
        out_type=jax.ShapeDtypeStruct((P, D), jnp.float32),
        scratch_types=[
            pltpu.VMEM((per_w,), jnp.int32),
            pltpu.VMEM((per_w, D), jnp.float32),
            pltpu.SemaphoreType.DMA,
        ],
    )
    def gather_kernel(table_hbm, idx_hbm, out_hbm, idx_v, rows_v, sem):
        wid = lax.axis_index("s") * info.num_cores + lax.axis_index("c")
        base = wid * per_w
        pltpu.sync_copy(idx_hbm.at[pl.ds(base, per_w)], idx_v)
        pltpu.async_copy(table_hbm.at[idx_v], rows_v, sem).wait()
        pltpu.sync_copy(rows_v, out_hbm.at[pl.ds(base, per_w)])

    return gather_kernel(table, flat_idx)


def _fused_body(rows_ref, x_ref, q_ref, wkv_ref, w1_ref, b1_ref, w2_ref,
                b2_ref, lnw_ref, lnb_ref, idxs_ref, wproj_ref, bproj_ref,
                out_ref, k_s, v_s):
    lnw = lnw_ref[...]
    lnb = lnb_ref[...]
    wkv = wkv_ref[...]

    def norm(xr):
        mu = jnp.mean(xr, axis=1, keepdims=True)
        xc = xr - mu
        var = jnp.mean(xc * xc, axis=1, keepdims=True)
        return xc * lax.rsqrt(var + _EPS) * lnw + lnb

    na = norm(rows_ref[0:_M])
    nb = norm(rows_ref[_IDX_OFF:_IDX_OFF + _M])
    kvg = lax.dot_general(na, wkv, (((1,), (1,)), ((), ())),
                          preferred_element_type=jnp.float32)
    k_idx = lax.dot_general(nb, wkv[:_C, :], (((1,), (1,)), ((), ())),
                            preferred_element_type=jnp.float32)

    q = q_ref[...]
    outs = []
    for h in range(_H):
        sl = slice(h * _Dh, (h + 1) * _Dh)
        k_h = kvg[:, h * _Dh:(h + 1) * _Dh]
        v_h = kvg[:, _C + h * _Dh: _C + (h + 1) * _Dh]
        l = lax.dot_general(q[:, sl], k_h, (((1,), (1,)), ((), ())),
                            preferred_element_type=jnp.float32) * _SCALE
        m_ = jnp.max(l, axis=1, keepdims=True)
        p = jnp.exp(l - m_)
        s_ = jnp.sum(p, axis=1, keepdims=True)
        outs.append(lax.dot_general(p, v_h, (((1,), (0,)), ((), ())),
                                    preferred_element_type=jnp.float32) / s_)
    q_att = jnp.concatenate(outs, axis=1)

    hdn = lax.dot_general(q_att, w1_ref[...], (((1,), (1,)), ((), ())),
                          preferred_element_type=jnp.float32) + b1_ref[...]
    hdn = 0.5 * hdn * (1.0 + lax.erf(hdn * (2.0 ** -0.5)))
    qc = lax.dot_general(hdn, w2_ref[...], (((1,), (1,)), ((), ())),
                         preferred_element_type=jnp.float32) + b2_ref[...]

    sc = lax.dot_general(qc, k_idx, (((1,), (1,)), ((), ())),
                         preferred_element_type=jnp.float32)
    rmax = jnp.max(sc, axis=1, keepdims=True)
    jj = lax.broadcasted_iota(jnp.int32, (_M, _M), 1)
    cand = jnp.where(sc == rmax, jj, _M)
    cent = jnp.min(cand, axis=1, keepdims=True)
    c = jnp.sum(jnp.where(jj == cent, idxs_ref[...], 0),
                axis=1, keepdims=True)

    xb = x_ref[0]
    mu = jnp.mean(xb, axis=1, keepdims=True)
    xc = xb - mu
    var = jnp.mean(xc * xc, axis=1, keepdims=True)
    nx = xc * lax.rsqrt(var + _EPS) * lnw + lnb
    k_s[...] = lax.dot_general(nx, wkv[:_C, :], (((1,), (1,)), ((), ())),
                               preferred_element_type=jnp.float32
                               ).astype(jnp.bfloat16)
    v_s[...] = lax.dot_general(nx, wkv[_C:, :], (((1,), (1,)), ((), ())),
                               preferred_element_type=jnp.float32
                               ).astype(jnp.bfloat16)

    n_row = lax.broadcasted_iota(jnp.int32, (1, _N), 1)
    a_row = lax.rem(n_row + 6, _PER)
    e = a_row - lax.rem(c, _PER)
    f = jnp.where(e < 0, e + _PER, e)
    d = n_row - c
    valid = (f <= 12) & (jnp.abs(d) <= _RAD)
    valid = valid | ((n_row == 0) & (c <= _RAD))
    valid = valid | ((n_row == _N - 1) & (c >= _N - 1 - _RAD))
    bias = jnp.where(valid, 0.0, _NEG)

    qcs = (qc * _SCALE).astype(jnp.bfloat16)
    outs = []
    for h in range(_H):
        k_h = k_s[:, h * _Dh:(h + 1) * _Dh]
        v_h = v_s[:, h * _Dh:(h + 1) * _Dh]
        l = lax.dot_general(qcs[:, h * _Dh:(h + 1) * _Dh], k_h,
                            (((1,), (1,)), ((), ())),
                            preferred_element_type=jnp.float32) + bias
        m_ = jnp.max(l, axis=1, keepdims=True)
        p = jnp.exp(l - m_)
        s_ = jnp.sum(p, axis=1, keepdims=True)
        pb = p.astype(jnp.bfloat16)
        outs.append(lax.dot_general(pb, v_h, (((1,), (0,)), ((), ())),
                                    preferred_element_type=jnp.float32) / s_)
    att = jnp.concatenate(outs, axis=1)
    out_ref[0] = lax.dot_general(att, wproj_ref[...], (((1,), (1,)), ((), ())),
                                 preferred_element_type=jnp.float32) + bproj_ref[...]


def kernel(x, q_param, W_kv, W_proj, b_proj, ln_w, ln_b,
           mlp_w1, mlp_b1, mlp_w2, mlp_b2, perm1, idxs):
    Bs, Ns, Cs = x.shape

    chunk_idx = jnp.zeros((_CHUNK,), jnp.int32)
    chunk_idx = chunk_idx.at[0:_M].set(perm1.astype(jnp.int32))
    chunk_idx = chunk_idx.at[_IDX_OFF:_IDX_OFF + _M].set(idxs.astype(jnp.int32))
    flat = (jnp.arange(Bs, dtype=jnp.int32)[:, None] * Ns
            + chunk_idx[None, :]).reshape(-1)
    rows = _sc_gather_rows(x.reshape(Bs * Ns, Cs), flat)

    q2 = q_param.reshape(_M, Cs)
    b1 = mlp_b1.reshape(1, 2 * Cs)
    b2 = mlp_b2.reshape(1, Cs)
    lnw = ln_w.reshape(1, Cs)
    lnb = ln_b.reshape(1, Cs)
    bp = b_proj.reshape(1, Cs)
    idxs_row = idxs.reshape(1, _M).astype(jnp.int32)

    out = pl.pallas_call(
        _fused_body,
        grid=(Bs,),
        in_specs=[
            pl.BlockSpec((_CHUNK, Cs), lambda b: (b, 0)),
            pl.BlockSpec((1, Ns, Cs), lambda b: (b, 0, 0)),
            pl.BlockSpec((_M, Cs), lambda b: (0, 0)),
            pl.BlockSpec((2 * Cs, Cs), lambda b: (0, 0)),
            pl.BlockSpec((2 * Cs, Cs), lambda b: (0, 0)),
            pl.BlockSpec((1, 2 * Cs), lambda b: (0, 0)),
            pl.BlockSpec((Cs, 2 * Cs), lambda b: (0, 0)),
            pl.BlockSpec((1, Cs), lambda b: (0, 0)),
            pl.BlockSpec((1, Cs), lambda b: (0, 0)),
            pl.BlockSpec((1, Cs), lambda b: (0, 0)),
            pl.BlockSpec((1, _M), lambda b: (0, 0)),
            pl.BlockSpec((Cs, Cs), lambda b: (0, 0)),
            pl.BlockSpec((1, Cs), lambda b: (0, 0)),
        ],
        out_specs=pl.BlockSpec((1, _M, Cs), lambda b: (b, 0, 0)),
        out_shape=jax.ShapeDtypeStruct((Bs, _M, Cs), jnp.float32),
        scratch_shapes=[pltpu.VMEM((Ns, Cs), jnp.bfloat16),
                        pltpu.VMEM((Ns, Cs), jnp.bfloat16)],
    )(rows, x, q2, W_kv, mlp_w1, b1, mlp_w2, b2, lnw, lnb, idxs_row,
      W_proj, bp)
    return out
